# baseline (speedup 1.0000x reference)
import functools

import jax
import jax.numpy as jnp
from jax import lax
from jax.experimental import pallas as pl
from jax.experimental.pallas import tpu as pltpu
from jax.experimental.pallas import tpu_sc as plsc

B = 16384
SEQ = 200
VOCAB = 500
DIM = 10
VOCAB_PAD = 512
NUM_CORES = 2
NUM_SUBCORES = 16
NW = NUM_CORES * NUM_SUBCORES
ROWS_PER_W = B // NW
CHUNK = 64
NCHUNK = ROWS_PER_W // CHUNK
AUX_LEN = VOCAB * DIM + 2 * DIM + 2


def _bf16_round(x):
    u = plsc.bitcast(x, jnp.int32)
    rbit = lax.shift_right_logical(u, 16) & 1
    u = (u + 0x7FFF + rbit) & jnp.int32(-65536)
    return plsc.bitcast(u, jnp.float32)


def _rsqrt(x):
    i = plsc.bitcast(x, jnp.int32)
    i = 0x5F3759DF - lax.shift_right_arithmetic(i, 1)
    r = plsc.bitcast(i, jnp.float32)
    for _ in range(4):
        r = r * (1.5 - 0.5 * x * r * r)
    return r


@functools.partial(
    pl.kernel,
    out_type=jax.ShapeDtypeStruct((NUM_CORES, 16), jnp.float32),
    mesh=plsc.VectorSubcoreMesh(core_axis_name="c", subcore_axis_name="s"),
    compiler_params=pltpu.CompilerParams(needs_layout_passes=False),
    scratch_types=[
        pltpu.VMEM((CHUNK, SEQ), jnp.int32),
        pltpu.VMEM((CHUNK, SEQ), jnp.int32),
        pltpu.VMEM((ROWS_PER_W,), jnp.int32),
        pltpu.VMEM((AUX_LEN,), jnp.float32),
        pltpu.VMEM((2 * VOCAB_PAD,), jnp.float32),
        pltpu.VMEM((16,), jnp.float32),
        pltpu.VMEM((16 * NUM_SUBCORES,), jnp.float32),
        pltpu.VMEM_SHARED((16 * NUM_SUBCORES,), jnp.float32),
        pltpu.SemaphoreType.DMA,
        pltpu.SemaphoreType.DMA,
    ],
)
def _sc_call(idx_hbm, lab_hbm, aux_hbm, out_hbm,
             buf0, buf1, lab_buf, aux_buf, tab_buf, stage,
             red_buf, shared, sem0, sem1):
    c = lax.axis_index("c")
    s = lax.axis_index("s")
    w = c * NUM_SUBCORES + s
    base_row = w * ROWS_PER_W

    def chunk_src(ch):
        return idx_hbm.at[pl.ds(base_row + ch * CHUNK, CHUNK)]

    pltpu.async_copy(chunk_src(0), buf0, sem0)
    pltpu.sync_copy(aux_hbm, aux_buf)
    pltpu.sync_copy(lab_hbm.at[pl.ds(base_row, ROWS_PER_W)], lab_buf)

    lane = lax.broadcasted_iota(jnp.int32, (16,), 0)

    def splat(i):
        return plsc.load_gather(aux_buf, [jnp.full((16,), i, jnp.int32)])

    w_rows = [[_bf16_round(splat(VOCAB * DIM + cc * DIM + d))
               for d in range(DIM)] for cc in range(2)]
    bias = [splat(VOCAB * DIM + 2 * DIM + cc) * (1.0 / SEQ) for cc in range(2)]

    for vc in range(VOCAB_PAD // 16):
        v16 = jnp.minimum(vc * 16 + lane, VOCAB - 1) * DIM
        e = [plsc.load_gather(aux_buf, [v16 + d]) for d in range(DIM)]
        n2 = e[0] * e[0]
        for d in range(1, DIM):
            n2 = n2 + e[d] * e[d]
        r = _rsqrt(n2)
        scale = jnp.where(n2 > 1.0, r - 1e-7 * (r * r), 1.0)
        for cc in range(2):
            acc = e[0] * w_rows[cc][0]
            for d in range(1, DIM):
                acc = acc + e[d] * w_rows[cc][d]
            tab_buf[pl.ds(cc * VOCAB_PAD + vc * 16, 16)] = (
                acc * scale + bias[cc])

    tail_mask = (lane >= 12 * 16 - (SEQ - 16)).astype(jnp.float32)
    zero = jnp.zeros((16,), jnp.float32)

    def compute(buf, ch, accs):
        def row_quad_step(p, accs):
            a0, a1, a2, a3 = accs
            for q in range(4):
                rl = 4 * p + q
                off = plsc.load_gather(
                    lab_buf,
                    [jnp.full((16,), ch * CHUNK + rl, jnp.int32)]) * VOCAB_PAD
                for v in range(12):
                    key = buf[rl, pl.ds(v * 16, 16)] + off
                    g = plsc.load_gather(tab_buf, [key])
                    if v % 4 == 0:
                        a0 = a0 + g
                    elif v % 4 == 1:
                        a1 = a1 + g
                    elif v % 4 == 2:
                        a2 = a2 + g
                    else:
                        a3 = a3 + g
                key = buf[rl, pl.ds(SEQ - 16, 16)] + off
                a0 = a0 + plsc.load_gather(tab_buf, [key]) * tail_mask
            return (a0, a1, a2, a3)

        return lax.fori_loop(0, CHUNK // 4, row_quad_step, accs)

    def outer(g, accs):
        c0 = 2 * g
        c1 = 2 * g + 1
        pltpu.make_async_copy(chunk_src(c0), buf0, sem0).wait()
        pltpu.async_copy(chunk_src(c1), buf1, sem1)
        accs = compute(buf0, c0, accs)
        pltpu.make_async_copy(chunk_src(c1), buf1, sem1).wait()
        nxt = jnp.minimum(c1 + 1, NCHUNK - 1)
        pltpu.async_copy(chunk_src(nxt), buf0, sem0)
        accs = compute(buf1, c1, accs)
        return accs

    a0, a1, a2, a3 = lax.fori_loop(
        0, NCHUNK // 2, outer, (zero, zero, zero, zero))
    pltpu.make_async_copy(chunk_src(NCHUNK - 1), buf0, sem0).wait()

    stage[...] = (a0 + a1) + (a2 + a3)
    pltpu.sync_copy(stage, shared.at[pl.ds(s * 16, 16)])
    plsc.subcore_barrier()

    @pl.when(s == 0)
    def _():
        pltpu.sync_copy(shared, red_buf)
        tot = zero
        for i in range(NUM_SUBCORES):
            tot = tot + red_buf[pl.ds(i * 16, 16)]
        stage[...] = jnp.broadcast_to(jnp.sum(tot) * (-1.0 / B), (16,))
        pltpu.sync_copy(stage, out_hbm.at[c])


def kernel(input, labels, emb_table, W, b):
    idx2d = input.astype(jnp.int32)
    labels32 = labels.astype(jnp.int32)
    aux = jnp.concatenate(
        [emb_table.reshape(-1), W.reshape(-1), b])
    partials = _sc_call(idx2d, labels32, aux)
    return partials[0, 0] + partials[1, 0]
